# baseline (speedup 1.0000x reference)
import functools

import jax
import jax.numpy as jnp
import numpy as np
from jax import lax
from jax.experimental import pallas as pl
from jax.experimental.pallas import tpu as pltpu
from jax.experimental.pallas import tpu_sc as plsc

NE = 65536
NN = 4096
DUMMY = NN
NR = 4104
NT = 32
TPW = (2 * NE) // NT
CH = 128
NCH = TPW // CH

_TINY = float(np.finfo(np.float32).tiny)


def _threefry_xor_bits(idx):
    k0 = jnp.int32(0)
    k1 = jnp.int32(42)
    k2 = jnp.int32(0 ^ 42 ^ 0x1BD11BDA)
    ks = [k0, k1, k2]
    rot = [[13, 15, 26, 6], [17, 29, 16, 24]]

    def rotl(x, r):
        return lax.shift_left(x, jnp.int32(r)) | lax.shift_right_logical(
            x, jnp.int32(32 - r))

    x0 = jnp.zeros_like(idx) + ks[0]
    x1 = idx + ks[1]
    for i in range(5):
        for r in rot[i % 2]:
            x0 = x0 + x1
            x1 = rotl(x1, r)
            x1 = x1 ^ x0
        x0 = x0 + ks[(i + 1) % 3]
        x1 = x1 + ks[(i + 2) % 3] + jnp.int32(i + 1)
    return x0 ^ x1


def _gumbel_from_idx(idx):
    bits = _threefry_xor_bits(idx)
    fb = lax.shift_right_logical(bits, jnp.int32(9)) | jnp.int32(0x3F800000)
    u = lax.bitcast_convert_type(fb, jnp.float32) - jnp.float32(1.0)
    u = jnp.maximum(jnp.float32(_TINY),
                    u * jnp.float32(1.0 - _TINY) + jnp.float32(_TINY))
    return -jnp.log(-jnp.log(u))


def _sample_body(l0_ref, l1_ref, src_ref, dst_ref, s0_ref, s1_ref):
    rows, cols = l0_ref.shape
    r = lax.broadcasted_iota(jnp.int32, (rows, cols), 0)
    c = lax.broadcasted_iota(jnp.int32, (rows, cols), 1)
    i = r * cols + c
    g0 = _gumbel_from_idx(2 * i)
    g1 = _gumbel_from_idx(2 * i + 1)
    l0 = l0_ref[...]
    l1 = l1_ref[...]
    m = jnp.maximum(l0, l1)
    e0 = jnp.exp(l0 - m)
    e1 = jnp.exp(l1 - m)
    s = e0 + e1
    z0 = jnp.log(e0 / s + jnp.float32(1e-20))
    z1 = jnp.log(e1 / s + jnp.float32(1e-20))
    keep = (z1 + g1) > (z0 + g0)
    s0_ref[...] = jnp.where(keep, src_ref[...], jnp.int32(DUMMY))
    s1_ref[...] = jnp.where(keep, dst_ref[...], jnp.int32(DUMMY))


@functools.lru_cache(maxsize=None)
def _make_sc_agg():
    D = 128
    mesh = plsc.VectorSubcoreMesh(core_axis_name="c", subcore_axis_name="s")
    zrows = NN // 16

    @functools.partial(
        pl.kernel,
        mesh=mesh,
        out_type=jax.ShapeDtypeStruct((2, NR, D), jnp.float32),
        scratch_types=[
            pltpu.VMEM((NCH, CH), jnp.int32),
            pltpu.VMEM((NCH, CH), jnp.int32),
            pltpu.VMEM((CH, D), jnp.float32),
            pltpu.VMEM((CH, D), jnp.float32),
            pltpu.VMEM((zrows, D), jnp.float32),
            pltpu.VMEM_SHARED((NR, D), jnp.float32),
            pltpu.SemaphoreType.DMA,
            pltpu.SemaphoreType.DMA,
            pltpu.SemaphoreType.DMA,
            pltpu.SemaphoreType.DMA,
        ],
    )
    def agg(table_h, gidx_h, sidx_h, out_h, gall_v, sall_v, rows0_v, rows1_v,
            zbuf_v, accum_s, sem0, sem1, ssem0, ssem1):
        cid = lax.axis_index("c")
        sid = lax.axis_index("s")
        wid = cid * 16 + sid

        pltpu.sync_copy(gidx_h.at[wid], gall_v)
        pltpu.sync_copy(sidx_h.at[wid], sall_v)

        def zrow(rr, _):
            for j in range(D // 16):
                zbuf_v[rr, pl.ds(j * 16, 16)] = jnp.zeros((16,), jnp.float32)
            return 0

        lax.fori_loop(0, zrows, zrow, 0)
        pltpu.sync_copy(zbuf_v, accum_s.at[pl.ds(sid * zrows, zrows)])
        plsc.subcore_barrier()

        def gather(g, rbuf, sem):
            pltpu.async_copy(table_h.at[gall_v.at[g]], rbuf, sem)

        def wait_gather(g, rbuf, sem):
            pltpu.make_async_copy(table_h.at[gall_v.at[g]], rbuf, sem).wait()

        def scatter(g, rbuf, ssem):
            pltpu.async_copy(rbuf, accum_s.at[sall_v.at[g]], ssem, add=True)

        def wait_scatter(g, rbuf, ssem):
            pltpu.make_async_copy(rbuf, accum_s.at[sall_v.at[g]],
                                  ssem).wait()

        gather(0, rows0_v, sem0)
        gather(1, rows1_v, sem1)

        def body2(j, _):
            g0 = 2 * j
            wait_gather(g0, rows0_v, sem0)
            scatter(g0, rows0_v, ssem0)
            wait_gather(g0 + 1, rows1_v, sem1)
            scatter(g0 + 1, rows1_v, ssem1)

            @pl.when(g0 + 2 < NCH)
            def _():
                wait_scatter(g0, rows0_v, ssem0)
                gather(g0 + 2, rows0_v, sem0)
                wait_scatter(g0 + 1, rows1_v, ssem1)
                gather(g0 + 3, rows1_v, sem1)

            return 0

        lax.fori_loop(0, NCH // 2, body2, 0)
        wait_scatter(NCH - 2, rows0_v, ssem0)
        wait_scatter(NCH - 1, rows1_v, ssem1)
        plsc.subcore_barrier()

        @pl.when(sid == 0)
        def _():
            pltpu.sync_copy(accum_s, out_h.at[cid])

    return agg


_M = 16384.0


def _dense1_body(p2_ref, x_ref, w1_ref, b1_ref, h_ref, deg_ref):
    psum = p2_ref[0:NN, :] + p2_ref[NR:NR + NN, :]
    col0 = psum[:, 0:1]
    cnt = jnp.floor(col0 * jnp.float32(1.0 / _M) + jnp.float32(0.5))
    sx0 = col0 - cnt * jnp.float32(_M)
    deg = cnt + jnp.float32(1.0)
    tot = psum + x_ref[...]
    lane = lax.broadcasted_iota(jnp.int32, tot.shape, 1)
    tot = jnp.where(lane == 0, sx0 + x_ref[:, 0:1], tot)
    norm = tot / deg
    h = jnp.dot(norm, w1_ref[...], preferred_element_type=jnp.float32)
    h_ref[...] = jnp.maximum(h + b1_ref[...], jnp.float32(0.0))
    deg_ref[...] = deg


def _dense2_body(q2_ref, h_ref, deg_ref, w2_ref, b2_ref, y_ref, loss_ref):
    tot = q2_ref[0:NN, :] + q2_ref[NR:NR + NN, :] + h_ref[...]
    norm = tot / deg_ref[...]
    ly = jnp.dot(norm, w2_ref[...], preferred_element_type=jnp.float32)
    ly = ly + b2_ref[...]
    m = jnp.max(ly, axis=1, keepdims=True)
    lse = m + jnp.log(jnp.sum(jnp.exp(ly - m), axis=1, keepdims=True))
    logp = ly - lse
    cls = lax.broadcasted_iota(jnp.int32, ly.shape, 1)
    picked = jnp.where(cls == y_ref[...], logp, jnp.float32(0.0))
    loss_ref[...] = (-jnp.sum(picked) / jnp.float32(NN)).reshape(1, 1)


def kernel(X, logit_E, Y, src, dst, W1, b1, W2, b2):
    l0 = logit_E[:, 0].reshape(512, 128)
    l1 = logit_E[:, 1].reshape(512, 128)
    srcb = src.astype(jnp.int32).reshape(512, 128)
    dstb = dst.astype(jnp.int32).reshape(512, 128)

    s0, s1 = pl.pallas_call(
        _sample_body,
        out_shape=[
            jax.ShapeDtypeStruct((512, 128), jnp.int32),
            jax.ShapeDtypeStruct((512, 128), jnp.int32),
        ],
    )(l0, l1, srcb, dstb)

    g_idx = jnp.concatenate(
        [dstb.reshape(-1), srcb.reshape(-1)]).reshape(NT, NCH, CH)
    s_idx = jnp.concatenate(
        [s0.reshape(-1), s1.reshape(-1)]).reshape(NT, NCH, CH)

    Xp = jnp.concatenate([X[:, 0:1] + jnp.float32(_M), X[:, 1:]], axis=1)
    p = _make_sc_agg()(Xp, g_idx, s_idx)

    h, deg = pl.pallas_call(
        _dense1_body,
        out_shape=[
            jax.ShapeDtypeStruct((NN, 128), jnp.float32),
            jax.ShapeDtypeStruct((NN, 1), jnp.float32),
        ],
    )(p.reshape(2 * NR, 128), X, W1, b1.reshape(1, 128))

    q = _make_sc_agg()(h, g_idx, s_idx)

    loss = pl.pallas_call(
        _dense2_body,
        out_shape=jax.ShapeDtypeStruct((1, 1), jnp.float32),
    )(q.reshape(2 * NR, 128), h, deg, W2, b2.reshape(1, 16),
      Y.astype(jnp.int32).reshape(NN, 1))

    return loss[0, 0]


# speedup vs baseline: 1.1262x; 1.1262x over previous
import functools

import jax
import jax.numpy as jnp
import numpy as np
from jax import lax
from jax.experimental import pallas as pl
from jax.experimental.pallas import tpu as pltpu
from jax.experimental.pallas import tpu_sc as plsc

NE = 65536
NN = 4096
DUMMY = NN
NR = 4104
NT = 32
TPW = (2 * NE) // NT
CH = 128
NCH = TPW // CH

_TINY = float(np.finfo(np.float32).tiny)


def _threefry_xor_bits(idx):
    k0 = jnp.int32(0)
    k1 = jnp.int32(42)
    k2 = jnp.int32(0 ^ 42 ^ 0x1BD11BDA)
    ks = [k0, k1, k2]
    rot = [[13, 15, 26, 6], [17, 29, 16, 24]]

    def rotl(x, r):
        return lax.shift_left(x, jnp.int32(r)) | lax.shift_right_logical(
            x, jnp.int32(32 - r))

    x0 = jnp.zeros_like(idx) + ks[0]
    x1 = idx + ks[1]
    for i in range(5):
        for r in rot[i % 2]:
            x0 = x0 + x1
            x1 = rotl(x1, r)
            x1 = x1 ^ x0
        x0 = x0 + ks[(i + 1) % 3]
        x1 = x1 + ks[(i + 2) % 3] + jnp.int32(i + 1)
    return x0 ^ x1


def _gumbel_from_idx(idx):
    bits = _threefry_xor_bits(idx)
    fb = lax.shift_right_logical(bits, jnp.int32(9)) | jnp.int32(0x3F800000)
    u = lax.bitcast_convert_type(fb, jnp.float32) - jnp.float32(1.0)
    u = jnp.maximum(jnp.float32(_TINY),
                    u * jnp.float32(1.0 - _TINY) + jnp.float32(_TINY))
    return -jnp.log(-jnp.log(u))


def _sample_body(l0_ref, l1_ref, src_ref, dst_ref, s0_ref, s1_ref):
    rows, cols = l0_ref.shape
    r = lax.broadcasted_iota(jnp.int32, (rows, cols), 0)
    c = lax.broadcasted_iota(jnp.int32, (rows, cols), 1)
    i = r * cols + c
    g0 = _gumbel_from_idx(2 * i)
    g1 = _gumbel_from_idx(2 * i + 1)
    l0 = l0_ref[...]
    l1 = l1_ref[...]
    m = jnp.maximum(l0, l1)
    e0 = jnp.exp(l0 - m)
    e1 = jnp.exp(l1 - m)
    s = e0 + e1
    z0 = jnp.log(e0 / s + jnp.float32(1e-20))
    z1 = jnp.log(e1 / s + jnp.float32(1e-20))
    keep = (z1 + g1) > (z0 + g0)
    s0_ref[...] = jnp.where(keep, src_ref[...], jnp.int32(DUMMY))
    s1_ref[...] = jnp.where(keep, dst_ref[...], jnp.int32(DUMMY))


@functools.lru_cache(maxsize=None)
def _make_sc_agg():
    D = 128
    mesh = plsc.VectorSubcoreMesh(core_axis_name="c", subcore_axis_name="s")
    zrows = NN // 16

    @functools.partial(
        pl.kernel,
        mesh=mesh,
        out_type=jax.ShapeDtypeStruct((2, NR, D), jnp.float32),
        scratch_types=[
            pltpu.VMEM((NCH, CH), jnp.int32),
            pltpu.VMEM((NCH, CH), jnp.int32),
            pltpu.VMEM((CH, D), jnp.float32),
            pltpu.VMEM((CH, D), jnp.float32),
            pltpu.VMEM((zrows, D), jnp.float32),
            pltpu.VMEM_SHARED((NR, D), jnp.float32),
            pltpu.SemaphoreType.DMA,
            pltpu.SemaphoreType.DMA,
        ],
    )
    def agg(table_h, gidx_h, sidx_h, out_h, gall_v, sall_v, rows0_v, rows1_v,
            zbuf_v, accum_s, sem0, sem1):
        cid = lax.axis_index("c")
        sid = lax.axis_index("s")
        wid = cid * 16 + sid

        pltpu.sync_copy(gidx_h.at[wid], gall_v)
        pltpu.sync_copy(sidx_h.at[wid], sall_v)

        def zrow(rr, _):
            for j in range(D // 16):
                zbuf_v[rr, pl.ds(j * 16, 16)] = jnp.zeros((16,), jnp.float32)
            return 0

        lax.fori_loop(0, zrows, zrow, 0)
        pltpu.sync_copy(zbuf_v, accum_s.at[pl.ds(sid * zrows, zrows)])
        plsc.subcore_barrier()

        def gather(g, rbuf, sem):
            pltpu.async_copy(table_h.at[gall_v.at[g]], rbuf, sem)

        def drain_scatter(g, rbuf, sem):
            pltpu.make_async_copy(table_h.at[gall_v.at[g]], rbuf, sem).wait()
            pltpu.sync_copy(rbuf, accum_s.at[sall_v.at[g]], add=True)

        gather(0, rows0_v, sem0)

        def body2(j, _):
            g0 = 2 * j
            gather(g0 + 1, rows1_v, sem1)
            drain_scatter(g0, rows0_v, sem0)

            @pl.when(g0 + 2 < NCH)
            def _():
                gather(g0 + 2, rows0_v, sem0)

            drain_scatter(g0 + 1, rows1_v, sem1)
            return 0

        lax.fori_loop(0, NCH // 2, body2, 0)
        plsc.subcore_barrier()

        @pl.when(sid == 0)
        def _():
            pltpu.sync_copy(accum_s, out_h.at[cid])

    return agg


_M = 16384.0


def _dense1_body(p2_ref, x_ref, w1_ref, b1_ref, h_ref, deg_ref):
    psum = p2_ref[0:NN, :] + p2_ref[NR:NR + NN, :]
    col0 = psum[:, 0:1]
    cnt = jnp.floor(col0 * jnp.float32(1.0 / _M) + jnp.float32(0.5))
    sx0 = col0 - cnt * jnp.float32(_M)
    deg = cnt + jnp.float32(1.0)
    tot = psum + x_ref[...]
    lane = lax.broadcasted_iota(jnp.int32, tot.shape, 1)
    tot = jnp.where(lane == 0, sx0 + x_ref[:, 0:1], tot)
    norm = tot / deg
    h = jnp.dot(norm, w1_ref[...], preferred_element_type=jnp.float32)
    h_ref[...] = jnp.maximum(h + b1_ref[...], jnp.float32(0.0))
    deg_ref[...] = deg


def _dense2_body(q2_ref, h_ref, deg_ref, w2_ref, b2_ref, y_ref, loss_ref):
    tot = q2_ref[0:NN, :] + q2_ref[NR:NR + NN, :] + h_ref[...]
    norm = tot / deg_ref[...]
    ly = jnp.dot(norm, w2_ref[...], preferred_element_type=jnp.float32)
    ly = ly + b2_ref[...]
    m = jnp.max(ly, axis=1, keepdims=True)
    lse = m + jnp.log(jnp.sum(jnp.exp(ly - m), axis=1, keepdims=True))
    logp = ly - lse
    cls = lax.broadcasted_iota(jnp.int32, ly.shape, 1)
    picked = jnp.where(cls == y_ref[...], logp, jnp.float32(0.0))
    loss_ref[...] = (-jnp.sum(picked) / jnp.float32(NN)).reshape(1, 1)


def kernel(X, logit_E, Y, src, dst, W1, b1, W2, b2):
    l0 = logit_E[:, 0].reshape(512, 128)
    l1 = logit_E[:, 1].reshape(512, 128)
    srcb = src.astype(jnp.int32).reshape(512, 128)
    dstb = dst.astype(jnp.int32).reshape(512, 128)

    s0, s1 = pl.pallas_call(
        _sample_body,
        out_shape=[
            jax.ShapeDtypeStruct((512, 128), jnp.int32),
            jax.ShapeDtypeStruct((512, 128), jnp.int32),
        ],
    )(l0, l1, srcb, dstb)

    g_idx = jnp.concatenate(
        [dstb.reshape(-1), srcb.reshape(-1)]).reshape(NT, NCH, CH)
    s_idx = jnp.concatenate(
        [s0.reshape(-1), s1.reshape(-1)]).reshape(NT, NCH, CH)

    Xp = jnp.concatenate([X[:, 0:1] + jnp.float32(_M), X[:, 1:]], axis=1)
    p = _make_sc_agg()(Xp, g_idx, s_idx)

    h, deg = pl.pallas_call(
        _dense1_body,
        out_shape=[
            jax.ShapeDtypeStruct((NN, 128), jnp.float32),
            jax.ShapeDtypeStruct((NN, 1), jnp.float32),
        ],
    )(p.reshape(2 * NR, 128), X, W1, b1.reshape(1, 128))

    q = _make_sc_agg()(h, g_idx, s_idx)

    loss = pl.pallas_call(
        _dense2_body,
        out_shape=jax.ShapeDtypeStruct((1, 1), jnp.float32),
    )(q.reshape(2 * NR, 128), h, deg, W2, b2.reshape(1, 16),
      Y.astype(jnp.int32).reshape(NN, 1))

    return loss[0, 0]


# speedup vs baseline: 1.1563x; 1.0267x over previous
import functools

import jax
import jax.numpy as jnp
import numpy as np
from jax import lax
from jax.experimental import pallas as pl
from jax.experimental.pallas import tpu as pltpu
from jax.experimental.pallas import tpu_sc as plsc

NE = 65536
NN = 4096
DUMMY = NN
NR = 4104
NT = 32
TPW = (2 * NE) // NT
CH = 128
NCH = TPW // CH

_TINY = float(np.finfo(np.float32).tiny)
_M = 16384.0


def _threefry_xor_bits(idx):
    k0 = jnp.int32(0)
    k1 = jnp.int32(42)
    k2 = jnp.int32(0 ^ 42 ^ 0x1BD11BDA)
    ks = [k0, k1, k2]
    rot = [[13, 15, 26, 6], [17, 29, 16, 24]]

    def rotl(x, r):
        return lax.shift_left(x, jnp.int32(r)) | lax.shift_right_logical(
            x, jnp.int32(32 - r))

    x0 = jnp.zeros_like(idx) + ks[0]
    x1 = idx + ks[1]
    for i in range(5):
        for r in rot[i % 2]:
            x0 = x0 + x1
            x1 = rotl(x1, r)
            x1 = x1 ^ x0
        x0 = x0 + ks[(i + 1) % 3]
        x1 = x1 + ks[(i + 2) % 3] + jnp.int32(i + 1)
    return x0 ^ x1


def _gumbel_from_idx(idx):
    bits = _threefry_xor_bits(idx)
    fb = lax.shift_right_logical(bits, jnp.int32(9)) | jnp.int32(0x3F800000)
    u = lax.bitcast_convert_type(fb, jnp.float32) - jnp.float32(1.0)
    u = jnp.maximum(jnp.float32(_TINY),
                    u * jnp.float32(1.0 - _TINY) + jnp.float32(_TINY))
    return -jnp.log(-jnp.log(u))


def _sample_body(l0_ref, l1_ref, src_ref, dst_ref, x_ref, g_ref, s_ref,
                 xp_ref):
    rows, cols = l0_ref.shape
    r = lax.broadcasted_iota(jnp.int32, (rows, cols), 0)
    c = lax.broadcasted_iota(jnp.int32, (rows, cols), 1)
    i = r * cols + c
    g0 = _gumbel_from_idx(2 * i)
    g1 = _gumbel_from_idx(2 * i + 1)
    l0 = l0_ref[...]
    l1 = l1_ref[...]
    m = jnp.maximum(l0, l1)
    e0 = jnp.exp(l0 - m)
    e1 = jnp.exp(l1 - m)
    s = e0 + e1
    z0 = jnp.log(e0 / s + jnp.float32(1e-20))
    z1 = jnp.log(e1 / s + jnp.float32(1e-20))
    keep = (z1 + g1) > (z0 + g0)
    src = src_ref[...]
    dst = dst_ref[...]
    g_ref[0:rows, :] = dst
    g_ref[rows:2 * rows, :] = src
    s_ref[0:rows, :] = jnp.where(keep, src, jnp.int32(DUMMY))
    s_ref[rows:2 * rows, :] = jnp.where(keep, dst, jnp.int32(DUMMY))
    x = x_ref[...]
    lane = lax.broadcasted_iota(jnp.int32, x.shape, 1)
    xp_ref[...] = jnp.where(lane == 0, x + jnp.float32(_M), x)


@functools.lru_cache(maxsize=None)
def _make_sc_agg():
    D = 128
    mesh = plsc.VectorSubcoreMesh(core_axis_name="c", subcore_axis_name="s")
    zrows = NN // 16

    @functools.partial(
        pl.kernel,
        mesh=mesh,
        out_type=jax.ShapeDtypeStruct((2, NR, D), jnp.float32),
        scratch_types=[
            pltpu.VMEM((NCH, CH), jnp.int32),
            pltpu.VMEM((NCH, CH), jnp.int32),
            pltpu.VMEM((CH, D), jnp.float32),
            pltpu.VMEM((CH, D), jnp.float32),
            pltpu.VMEM((zrows, D), jnp.float32),
            pltpu.VMEM_SHARED((NR, D), jnp.float32),
            pltpu.SemaphoreType.DMA,
            pltpu.SemaphoreType.DMA,
        ],
    )
    def agg(table_h, gidx_h, sidx_h, out_h, gall_v, sall_v, rows0_v, rows1_v,
            zbuf_v, accum_s, sem0, sem1):
        cid = lax.axis_index("c")
        sid = lax.axis_index("s")
        wid = cid * 16 + sid

        pltpu.sync_copy(gidx_h.at[wid], gall_v)
        pltpu.sync_copy(sidx_h.at[wid], sall_v)

        def zrow(rr, _):
            for j in range(D // 16):
                zbuf_v[rr, pl.ds(j * 16, 16)] = jnp.zeros((16,), jnp.float32)
            return 0

        lax.fori_loop(0, zrows, zrow, 0)
        pltpu.sync_copy(zbuf_v, accum_s.at[pl.ds(sid * zrows, zrows)])
        plsc.subcore_barrier()

        def gather(g, rbuf, sem):
            pltpu.async_copy(table_h.at[gall_v.at[g]], rbuf, sem)

        def drain_scatter(g, rbuf, sem):
            pltpu.make_async_copy(table_h.at[gall_v.at[g]], rbuf, sem).wait()
            pltpu.sync_copy(rbuf, accum_s.at[sall_v.at[g]], add=True)

        gather(0, rows0_v, sem0)

        def body2(j, _):
            g0 = 2 * j
            gather(g0 + 1, rows1_v, sem1)
            drain_scatter(g0, rows0_v, sem0)

            @pl.when(g0 + 2 < NCH)
            def _():
                gather(g0 + 2, rows0_v, sem0)

            drain_scatter(g0 + 1, rows1_v, sem1)
            return 0

        lax.fori_loop(0, NCH // 2, body2, 0)
        plsc.subcore_barrier()
        pltpu.sync_copy(accum_s.at[pl.ds(sid * zrows, zrows)],
                        out_h.at[cid, pl.ds(sid * zrows, zrows)])

    return agg


def _dense1_body(p2_ref, x_ref, w1_ref, b1_ref, h_ref, deg_ref):
    psum = p2_ref[0:NN, :] + p2_ref[NR:NR + NN, :]
    col0 = psum[:, 0:1]
    cnt = jnp.floor(col0 * jnp.float32(1.0 / _M) + jnp.float32(0.5))
    sx0 = col0 - cnt * jnp.float32(_M)
    deg = cnt + jnp.float32(1.0)
    tot = psum + x_ref[...]
    lane = lax.broadcasted_iota(jnp.int32, tot.shape, 1)
    tot = jnp.where(lane == 0, sx0 + x_ref[:, 0:1], tot)
    norm = tot / deg
    h = jnp.dot(norm, w1_ref[...], preferred_element_type=jnp.float32)
    h_ref[...] = jnp.maximum(h + b1_ref[...], jnp.float32(0.0))
    deg_ref[...] = deg


def _dense2_body(q2_ref, h_ref, deg_ref, w2_ref, b2_ref, y_ref, loss_ref):
    tot = q2_ref[0:NN, :] + q2_ref[NR:NR + NN, :] + h_ref[...]
    norm = tot / deg_ref[...]
    ly = jnp.dot(norm, w2_ref[...], preferred_element_type=jnp.float32)
    ly = ly + b2_ref[...]
    m = jnp.max(ly, axis=1, keepdims=True)
    lse = m + jnp.log(jnp.sum(jnp.exp(ly - m), axis=1, keepdims=True))
    logp = ly - lse
    cls = lax.broadcasted_iota(jnp.int32, ly.shape, 1)
    picked = jnp.where(cls == y_ref[...], logp, jnp.float32(0.0))
    loss_ref[...] = (-jnp.sum(picked) / jnp.float32(NN)).reshape(1, 1)


def kernel(X, logit_E, Y, src, dst, W1, b1, W2, b2):
    l0 = logit_E[:, 0].reshape(512, 128)
    l1 = logit_E[:, 1].reshape(512, 128)
    srcb = src.astype(jnp.int32).reshape(512, 128)
    dstb = dst.astype(jnp.int32).reshape(512, 128)

    g_all, s_all, Xp = pl.pallas_call(
        _sample_body,
        out_shape=[
            jax.ShapeDtypeStruct((1024, 128), jnp.int32),
            jax.ShapeDtypeStruct((1024, 128), jnp.int32),
            jax.ShapeDtypeStruct((NN, 128), jnp.float32),
        ],
    )(l0, l1, srcb, dstb, X)

    g_idx = g_all.reshape(NT, NCH, CH)
    s_idx = s_all.reshape(NT, NCH, CH)

    p = _make_sc_agg()(Xp, g_idx, s_idx)

    h, deg = pl.pallas_call(
        _dense1_body,
        out_shape=[
            jax.ShapeDtypeStruct((NN, 128), jnp.float32),
            jax.ShapeDtypeStruct((NN, 1), jnp.float32),
        ],
    )(p.reshape(2 * NR, 128), X, W1, b1.reshape(1, 128))

    q = _make_sc_agg()(h, g_idx, s_idx)

    loss = pl.pallas_call(
        _dense2_body,
        out_shape=jax.ShapeDtypeStruct((1, 1), jnp.float32),
    )(q.reshape(2 * NR, 128), h, deg, W2, b2.reshape(1, 16),
      Y.astype(jnp.int32).reshape(NN, 1))

    return loss[0, 0]


# speedup vs baseline: 1.2022x; 1.0397x over previous
import functools

import jax
import jax.numpy as jnp
import numpy as np
from jax import lax
from jax.experimental import pallas as pl
from jax.experimental.pallas import tpu as pltpu
from jax.experimental.pallas import tpu_sc as plsc

NE = 65536
NN = 4096
DUMMY = NN
NR = 4104
NT = 32
TPW = (2 * NE) // NT
CH = 128
NCH = TPW // CH

_TINY = float(np.finfo(np.float32).tiny)
_M = 16384.0


def _threefry_xor_bits(idx):
    k0 = jnp.int32(0)
    k1 = jnp.int32(42)
    k2 = jnp.int32(0 ^ 42 ^ 0x1BD11BDA)
    ks = [k0, k1, k2]
    rot = [[13, 15, 26, 6], [17, 29, 16, 24]]

    def rotl(x, r):
        return lax.shift_left(x, jnp.int32(r)) | lax.shift_right_logical(
            x, jnp.int32(32 - r))

    x0 = jnp.zeros_like(idx) + ks[0]
    x1 = idx + ks[1]
    for i in range(5):
        for r in rot[i % 2]:
            x0 = x0 + x1
            x1 = rotl(x1, r)
            x1 = x1 ^ x0
        x0 = x0 + ks[(i + 1) % 3]
        x1 = x1 + ks[(i + 2) % 3] + jnp.int32(i + 1)
    return x0 ^ x1


def _gumbel_from_idx(idx):
    bits = _threefry_xor_bits(idx)
    fb = lax.shift_right_logical(bits, jnp.int32(9)) | jnp.int32(0x3F800000)
    u = lax.bitcast_convert_type(fb, jnp.float32) - jnp.float32(1.0)
    u = jnp.maximum(jnp.float32(_TINY),
                    u * jnp.float32(1.0 - _TINY) + jnp.float32(_TINY))
    return -jnp.log(-jnp.log(u))


def _sample_body(l0_ref, l1_ref, src_ref, dst_ref, x_ref, g_ref, s_ref,
                 xp_ref):
    rows, cols = l0_ref.shape
    r = lax.broadcasted_iota(jnp.int32, (rows, cols), 0)
    c = lax.broadcasted_iota(jnp.int32, (rows, cols), 1)
    i = r * cols + c
    g0 = _gumbel_from_idx(2 * i)
    g1 = _gumbel_from_idx(2 * i + 1)
    l0 = l0_ref[...]
    l1 = l1_ref[...]
    m = jnp.maximum(l0, l1)
    e0 = jnp.exp(l0 - m)
    e1 = jnp.exp(l1 - m)
    s = e0 + e1
    z0 = jnp.log(e0 / s + jnp.float32(1e-20))
    z1 = jnp.log(e1 / s + jnp.float32(1e-20))
    keep = (z1 + g1) > (z0 + g0)
    src = src_ref[...]
    dst = dst_ref[...]
    g_ref[0:rows, :] = dst
    g_ref[rows:2 * rows, :] = src
    s_ref[0:rows, :] = jnp.where(keep, src, jnp.int32(DUMMY))
    s_ref[rows:2 * rows, :] = jnp.where(keep, dst, jnp.int32(DUMMY))
    x = x_ref[...]
    lane = lax.broadcasted_iota(jnp.int32, x.shape, 1)
    xp_ref[...] = jnp.where(lane == 0, x + jnp.float32(_M), x)


@functools.lru_cache(maxsize=None)
def _make_sc_agg():
    D = 128
    mesh = plsc.VectorSubcoreMesh(core_axis_name="c", subcore_axis_name="s")
    zrows = NN // 16

    @functools.partial(
        pl.kernel,
        mesh=mesh,
        out_type=jax.ShapeDtypeStruct((2, NR, D), jnp.float32),
        scratch_types=[
            pltpu.VMEM((NCH, CH), jnp.int32),
            pltpu.VMEM((NCH, CH), jnp.int32),
            pltpu.VMEM((CH, D), jnp.float32),
            pltpu.VMEM((CH, D), jnp.float32),
            pltpu.VMEM((zrows, D), jnp.float32),
            pltpu.VMEM_SHARED((NR, D), jnp.float32),
            pltpu.SemaphoreType.DMA,
            pltpu.SemaphoreType.DMA,
        ],
    )
    def agg(table_h, gidx_h, sidx_h, out_h, gall_v, sall_v, rows0_v, rows1_v,
            zbuf_v, accum_s, sem0, sem1):
        cid = lax.axis_index("c")
        sid = lax.axis_index("s")
        wid = cid * 16 + sid

        pltpu.async_copy(gidx_h.at[wid], gall_v, sem0)
        pltpu.async_copy(sidx_h.at[wid], sall_v, sem1)

        def zrow(rr, _):
            for j in range(D // 16):
                zbuf_v[rr, pl.ds(j * 16, 16)] = jnp.zeros((16,), jnp.float32)
            return 0

        lax.fori_loop(0, zrows, zrow, 0)
        pltpu.make_async_copy(gidx_h.at[wid], gall_v, sem0).wait()
        pltpu.make_async_copy(sidx_h.at[wid], sall_v, sem1).wait()

        def gather(g, rbuf, sem):
            pltpu.async_copy(table_h.at[gall_v.at[g]], rbuf, sem)

        def drain_scatter(g, rbuf, sem):
            pltpu.make_async_copy(table_h.at[gall_v.at[g]], rbuf, sem).wait()
            pltpu.sync_copy(rbuf, accum_s.at[sall_v.at[g]], add=True)

        gather(0, rows0_v, sem0)
        gather(1, rows1_v, sem1)
        pltpu.sync_copy(zbuf_v, accum_s.at[pl.ds(sid * zrows, zrows)])
        plsc.subcore_barrier()

        def body2(j, _):
            g0 = 2 * j
            drain_scatter(g0, rows0_v, sem0)

            @pl.when(g0 + 2 < NCH)
            def _():
                gather(g0 + 2, rows0_v, sem0)

            drain_scatter(g0 + 1, rows1_v, sem1)

            @pl.when(g0 + 3 < NCH)
            def _():
                gather(g0 + 3, rows1_v, sem1)

            return 0

        lax.fori_loop(0, NCH // 2, body2, 0)
        plsc.subcore_barrier()
        pltpu.sync_copy(accum_s.at[pl.ds(sid * zrows, zrows)],
                        out_h.at[cid, pl.ds(sid * zrows, zrows)])

    return agg


def _dense1_body(p2_ref, x_ref, w1_ref, b1_ref, h_ref, deg_ref):
    psum = p2_ref[0:NN, :] + p2_ref[NR:NR + NN, :]
    col0 = psum[:, 0:1]
    cnt = jnp.floor(col0 * jnp.float32(1.0 / _M) + jnp.float32(0.5))
    sx0 = col0 - cnt * jnp.float32(_M)
    deg = cnt + jnp.float32(1.0)
    tot = psum + x_ref[...]
    lane = lax.broadcasted_iota(jnp.int32, tot.shape, 1)
    tot = jnp.where(lane == 0, sx0 + x_ref[:, 0:1], tot)
    norm = tot / deg
    h = jnp.dot(norm, w1_ref[...], preferred_element_type=jnp.float32)
    h_ref[...] = jnp.maximum(h + b1_ref[...], jnp.float32(0.0))
    deg_ref[...] = deg


def _dense2_body(q2_ref, h_ref, deg_ref, w2_ref, b2_ref, y_ref, loss_ref):
    tot = q2_ref[0:NN, :] + q2_ref[NR:NR + NN, :] + h_ref[...]
    norm = tot / deg_ref[...]
    ly = jnp.dot(norm, w2_ref[...], preferred_element_type=jnp.float32)
    ly = ly + b2_ref[...]
    m = jnp.max(ly, axis=1, keepdims=True)
    lse = m + jnp.log(jnp.sum(jnp.exp(ly - m), axis=1, keepdims=True))
    logp = ly - lse
    cls = lax.broadcasted_iota(jnp.int32, ly.shape, 1)
    picked = jnp.where(cls == y_ref[...], logp, jnp.float32(0.0))
    loss_ref[...] = (-jnp.sum(picked) / jnp.float32(NN)).reshape(1, 1)


def kernel(X, logit_E, Y, src, dst, W1, b1, W2, b2):
    l0 = logit_E[:, 0].reshape(512, 128)
    l1 = logit_E[:, 1].reshape(512, 128)
    srcb = src.astype(jnp.int32).reshape(512, 128)
    dstb = dst.astype(jnp.int32).reshape(512, 128)

    g_all, s_all, Xp = pl.pallas_call(
        _sample_body,
        out_shape=[
            jax.ShapeDtypeStruct((1024, 128), jnp.int32),
            jax.ShapeDtypeStruct((1024, 128), jnp.int32),
            jax.ShapeDtypeStruct((NN, 128), jnp.float32),
        ],
    )(l0, l1, srcb, dstb, X)

    g_idx = g_all.reshape(NT, NCH, CH)
    s_idx = s_all.reshape(NT, NCH, CH)

    p = _make_sc_agg()(Xp, g_idx, s_idx)

    h, deg = pl.pallas_call(
        _dense1_body,
        out_shape=[
            jax.ShapeDtypeStruct((NN, 128), jnp.float32),
            jax.ShapeDtypeStruct((NN, 1), jnp.float32),
        ],
    )(p.reshape(2 * NR, 128), X, W1, b1.reshape(1, 128))

    q = _make_sc_agg()(h, g_idx, s_idx)

    loss = pl.pallas_call(
        _dense2_body,
        out_shape=jax.ShapeDtypeStruct((1, 1), jnp.float32),
    )(q.reshape(2 * NR, 128), h, deg, W2, b2.reshape(1, 16),
      Y.astype(jnp.int32).reshape(NN, 1))

    return loss[0, 0]
